# baseline (speedup 1.0000x reference)
import functools

import jax
import jax.numpy as jnp
from jax import lax
from jax.experimental import pallas as pl
from jax.experimental.pallas import tpu as pltpu
from jax.experimental.pallas import tpu_sc as plsc

N = 10000
E = 320000
D = 128
DE = 16
TD = 32
MD = 128
ED = 128

NW = 32
EPW = E // NW
B = 80
K = EPW // B
BB = 40
KB = EPW // BB
KC = 50
N_PAD = 10240
RPT = N_PAD // 16

f32 = jnp.float32

_S = (0.9999997069578552, -0.16666577198040858, 0.008332557998290352,
      -0.00019812572236973137, 2.7040473311964398e-06,
      -2.053408005585339e-08)
_TWO_PI_HI = 6.28125
_TWO_PI_LO = 0.0019353071795864769


def _fast_sin(x):
    k = jnp.round(x * (1.0 / 6.283185307179586))
    r = (x - k * _TWO_PI_HI) - k * _TWO_PI_LO
    r2 = r * r
    s = _S[5]
    for c in (_S[4], _S[3], _S[2], _S[1], _S[0]):
        s = s * r2 + c
    return s * r


def _edge_pre_body(et_ref, ea_ref, wef_ref, bmsg_ref, tw_ref, tb_ref,
                   c2_ref, lat_ref, mx_ref):
    i = pl.program_id(0)
    et = et_ref[0, 0, :]
    te = _fast_sin(et[:, None] * tw_ref[0, :][None, :] + tb_ref[0, :][None, :])
    eafeat = jnp.concatenate([ea_ref[...], te], axis=1)
    c2_ref[...] = (jnp.dot(eafeat, wef_ref[...], preferred_element_type=f32)
                   + bmsg_ref[0, :][None, :])

    @pl.when(i == 0)
    def _():
        mx_ref[0] = -jnp.inf

    mx_ref[0] = jnp.maximum(mx_ref[0], jnp.max(et))

    @pl.when(i == pl.num_programs(0) - 1)
    def _():
        lat_ref[0, 0] = mx_ref[0]


def _node_pre_body(x_ref, w1x_ref, b1_ref, p1_ref):
    p1_ref[...] = (jnp.dot(x_ref[...], w1x_ref[...], preferred_element_type=f32)
                   + b1_ref[0, :][None, :])


def _msg_body(g1_ref, ea_ref, w1e_ref, w2_ref, b2_ref, out_ref):
    h = jnp.maximum(
        g1_ref[...] + jnp.dot(ea_ref[...], w1e_ref[...],
                              preferred_element_type=f32), 0.0)
    out_ref[...] = jnp.maximum(
        jnp.dot(h, w2_ref[...], preferred_element_type=f32)
        + b2_ref[0, :][None, :], 0.0)


def _hist_body(src_ref, cnt_ref):
    i = pl.program_id(0)
    srcv = src_ref[0, 0, :]
    g = (srcv[:, None] // 128
         == lax.broadcasted_iota(jnp.int32, (1, N_PAD // 128), 1)).astype(f32)
    o = (srcv[:, None] % 128
         == lax.broadcasted_iota(jnp.int32, (1, 128), 1)).astype(f32)
    contrib = lax.dot_general(g, o, (((0,), (0,)), ((), ())),
                              preferred_element_type=f32)

    @pl.when(i == 0)
    def _():
        cnt_ref[...] = jnp.zeros_like(cnt_ref)

    cnt_ref[...] += contrib


def _gru_body(pa_ref, cnt_ref, x_ref, lat_ref, tw_ref, tb_ref, wi_ref, bi_ref,
              bh_ref, wma_ref, woa_ref, bout_ref, paug_ref, aout_ref):
    cnt = cnt_ref[0, 0, :]
    agg = (pa_ref[0] + pa_ref[1]) / jnp.maximum(cnt, 1.0)[:, None]
    gi = jnp.dot(agg, wi_ref[...], preferred_element_type=f32) + bi_ref[0][None, :]
    bh = bh_ref[0]
    r = jax.nn.sigmoid(gi[:, :MD] + bh[None, :MD])
    z = jax.nn.sigmoid(gi[:, MD:2 * MD] + bh[None, MD:2 * MD])
    nt = jnp.tanh(gi[:, 2 * MD:] + r * bh[None, 2 * MD:])
    mem = (1.0 - z) * nt
    gte = jnp.sin(lat_ref[0, 0] * tw_ref[0] + tb_ref[0])
    aug = jnp.concatenate(
        [x_ref[...], mem,
         jnp.broadcast_to(gte[None, :], (mem.shape[0], TD))], axis=1)
    paug_ref[...] = jnp.dot(aug, wma_ref[...], preferred_element_type=f32)
    aout_ref[...] = (jnp.dot(aug, woa_ref[...], preferred_element_type=f32)
                     + bout_ref[0][None, :])


def _final_body(pa_ref, aout_ref, wog_ref, wc1_ref, bc1_ref, wc2_ref, bc2_ref,
                out_ref):
    agg2 = pa_ref[0] + pa_ref[1]
    x = jnp.maximum(
        aout_ref[...] + jnp.dot(agg2, wog_ref[...], preferred_element_type=f32),
        0.0)
    h2 = jnp.maximum(
        jnp.dot(x, wc1_ref[...], preferred_element_type=f32)
        + bc1_ref[0][None, :], 0.0)
    out_ref[...] = (jnp.dot(h2, wc2_ref[...], preferred_element_type=f32)
                    + bc2_ref[0][None, :])


_MESH = plsc.VectorSubcoreMesh(core_axis_name="c", subcore_axis_name="s")


def _zero_accum(buf, accum, rs):
    nb = buf.shape[0]

    def zb(r, carry):
        for cc8 in range(8):
            buf[r, pl.ds(cc8 * 16, 16)] = jnp.zeros((16,), f32)
        return carry

    lax.fori_loop(0, nb, zb, 0)

    def cp(k, carry):
        off = pl.multiple_of(rs + k * nb, 8)
        pltpu.sync_copy(buf, accum.at[pl.ds(off, nb)])
        return carry

    lax.fori_loop(0, RPT // nb, cp, 0)


@functools.partial(
    pl.kernel, mesh=_MESH,
    out_type=jax.ShapeDtypeStruct((E, D), f32),
    scratch_types=[
        pltpu.VMEM((K, B), jnp.int32),
        pltpu.VMEM((B, D), f32),
        pltpu.VMEM((B, D), f32),
        pltpu.SemaphoreType.DMA,
        pltpu.SemaphoreType.DMA,
    ])
def _sc_gather(table_hbm, idx_hbm, out_hbm, idx_v, rows, rows1, sem, sem1):
    c = lax.axis_index("c")
    s = lax.axis_index("s")
    wid = s * 2 + c
    pltpu.sync_copy(idx_hbm.at[wid], idx_v)
    base = wid * EPW

    def body(t, carry):
        j0 = t * 2
        g0 = pltpu.make_async_copy(table_hbm.at[idx_v.at[j0]], rows, sem)
        g0.start()
        g1 = pltpu.make_async_copy(table_hbm.at[idx_v.at[j0 + 1]], rows1, sem1)
        g1.start()
        g0.wait()
        pltpu.sync_copy(rows, out_hbm.at[pl.ds(pl.multiple_of(base + j0 * B, 16), B)])
        g1.wait()
        pltpu.sync_copy(rows1, out_hbm.at[pl.ds(pl.multiple_of(base + (j0 + 1) * B, 16), B)])
        return carry

    lax.fori_loop(0, K // 2, body, 0)
    jt = K - 1
    pltpu.async_copy(table_hbm.at[idx_v.at[jt]], rows, sem).wait()
    pltpu.sync_copy(rows, out_hbm.at[pl.ds(pl.multiple_of(base + jt * B, 16), B)])


@functools.partial(
    pl.kernel, mesh=_MESH,
    out_type=jax.ShapeDtypeStruct((2, N_PAD, MD), f32),
    scratch_types=[
        pltpu.VMEM((K, B), jnp.int32),
        pltpu.VMEM((B, MD), f32),
        pltpu.VMEM((B, MD), f32),
        pltpu.VMEM_SHARED((N_PAD, MD), f32),
        pltpu.SemaphoreType.DMA,
        pltpu.SemaphoreType.DMA,
    ])
def _sc_scatter(msg_hbm, idx_hbm, out_hbm, idx_v, rows, rows1, accum, sem,
                sem1):
    c = lax.axis_index("c")
    s = lax.axis_index("s")
    wid = s * 2 + c
    rs = s * RPT
    _zero_accum(rows, accum, rs)
    plsc.subcore_barrier()
    pltpu.sync_copy(idx_hbm.at[wid], idx_v)
    base = wid * EPW

    def body(t, carry):
        j0 = t * 2
        off0 = pl.multiple_of(base + j0 * B, 16)
        off1 = pl.multiple_of(base + (j0 + 1) * B, 16)
        g0 = pltpu.make_async_copy(msg_hbm.at[pl.ds(off0, B)], rows, sem)
        g0.start()
        g1 = pltpu.make_async_copy(msg_hbm.at[pl.ds(off1, B)], rows1, sem1)
        g1.start()
        g0.wait()
        pltpu.sync_copy(rows, accum.at[idx_v.at[j0]], add=True)
        g1.wait()
        pltpu.sync_copy(rows1, accum.at[idx_v.at[j0 + 1]], add=True)
        return carry

    lax.fori_loop(0, K // 2, body, 0)
    jt = K - 1
    offt = pl.multiple_of(base + jt * B, 16)
    pltpu.async_copy(msg_hbm.at[pl.ds(offt, B)], rows, sem).wait()
    pltpu.sync_copy(rows, accum.at[idx_v.at[jt]], add=True)
    plsc.subcore_barrier()
    pltpu.sync_copy(accum.at[pl.ds(rs, RPT)], out_hbm.at[c, pl.ds(rs, RPT)])


@functools.partial(
    pl.kernel, mesh=_MESH,
    out_type=jax.ShapeDtypeStruct((2, N_PAD, ED), f32),
    scratch_types=[
        pltpu.VMEM((KC, BB), jnp.int32),
        pltpu.VMEM((KC, BB), jnp.int32),
        pltpu.VMEM((BB, ED), f32),
        pltpu.VMEM((BB, ED), f32),
        pltpu.VMEM((BB, ED), f32),
        pltpu.VMEM((BB, ED), f32),
        pltpu.VMEM_SHARED((N_PAD, ED), f32),
        pltpu.SemaphoreType.DMA,
        pltpu.SemaphoreType.DMA,
        pltpu.SemaphoreType.DMA,
        pltpu.SemaphoreType.DMA,
    ])
def _sc_edgeb(paug_hbm, c2_hbm, sidx_hbm, didx_hbm, out_hbm,
              sidx_v, didx_v, gbuf0, gbuf1, cbuf0, cbuf1, accum,
              gsem0, gsem1, csem0, csem1):
    c = lax.axis_index("c")
    s = lax.axis_index("s")
    wid = s * 2 + c
    rs = s * RPT
    _zero_accum(gbuf0, accum, rs)
    plsc.subcore_barrier()
    base = wid * EPW

    def _relu_add(gbuf, cbuf):
        def ew_row(r, carry):
            for cc8 in range(8):
                col = cc8 * 16
                v = gbuf[r, pl.ds(col, 16)] + cbuf[r, pl.ds(col, 16)]
                gbuf[r, pl.ds(col, 16)] = jnp.maximum(v, 0.0)
            return carry

        lax.fori_loop(0, BB, ew_row, 0)

    def outer(jj, carry):
        pltpu.sync_copy(sidx_hbm.at[wid, jj], sidx_v)
        pltpu.sync_copy(didx_hbm.at[wid, jj], didx_v)

        def body(t, carry2):
            j0 = t * 2
            off0 = pl.multiple_of(base + (jj * KC + j0) * BB, 8)
            off1 = pl.multiple_of(base + (jj * KC + j0 + 1) * BB, 8)
            c0 = pltpu.make_async_copy(c2_hbm.at[pl.ds(off0, BB)], cbuf0, csem0)
            c0.start()
            g0 = pltpu.make_async_copy(paug_hbm.at[sidx_v.at[j0]], gbuf0, gsem0)
            g0.start()
            c1 = pltpu.make_async_copy(c2_hbm.at[pl.ds(off1, BB)], cbuf1, csem1)
            c1.start()
            g1 = pltpu.make_async_copy(paug_hbm.at[sidx_v.at[j0 + 1]], gbuf1,
                                       gsem1)
            g1.start()
            g0.wait()
            c0.wait()
            _relu_add(gbuf0, cbuf0)
            pltpu.sync_copy(gbuf0, accum.at[didx_v.at[j0]], add=True)
            g1.wait()
            c1.wait()
            _relu_add(gbuf1, cbuf1)
            pltpu.sync_copy(gbuf1, accum.at[didx_v.at[j0 + 1]], add=True)
            return carry2

        lax.fori_loop(0, KC // 2, body, 0)
        return carry

    lax.fori_loop(0, KB // KC, outer, 0)
    plsc.subcore_barrier()
    pltpu.sync_copy(accum.at[pl.ds(rs, RPT)], out_hbm.at[c, pl.ds(rs, RPT)])


def kernel(node_features, edge_index, edge_attr, edge_times, time_w, time_b,
           W1, b1, W2, b2, Wi, bi, Wh, bh, Wmsg, bmsg, Wout, bout,
           Wc1, bc1, Wc2, bc2):
    del Wh
    BE = 2000
    BN = 2000
    nbe = E // BE
    nbn = N // BN
    AUG = D + MD + TD

    src = edge_index[0].astype(jnp.int32)
    dst = edge_index[1].astype(jnp.int32)
    src3 = src.reshape(NW, K, B)
    dst3 = dst.reshape(NW, K, B)
    src_tc = src.reshape(nbe, 1, BE)
    et3 = edge_times.reshape(nbe, 1, BE)
    tw2 = time_w.reshape(1, TD)
    tb2 = time_b.reshape(1, TD)

    W1x = W1[:D]
    W1e = W1[D + MD:]
    Wm_aug = Wmsg[:AUG]
    Wm_ef = Wmsg[AUG:]
    Wo_aug = Wout[:AUG]
    Wo_agg = Wout[AUG:]

    c2, latest = pl.pallas_call(
        _edge_pre_body,
        grid=(nbe,),
        in_specs=[
            pl.BlockSpec((1, 1, BE), lambda i: (i, 0, 0)),
            pl.BlockSpec((BE, DE), lambda i: (i, 0)),
            pl.BlockSpec((DE + TD, ED), lambda i: (0, 0)),
            pl.BlockSpec((1, ED), lambda i: (0, 0)),
            pl.BlockSpec((1, TD), lambda i: (0, 0)),
            pl.BlockSpec((1, TD), lambda i: (0, 0)),
        ],
        out_specs=[
            pl.BlockSpec((BE, ED), lambda i: (i, 0)),
            pl.BlockSpec(memory_space=pltpu.SMEM),
        ],
        out_shape=[
            jax.ShapeDtypeStruct((E, ED), f32),
            jax.ShapeDtypeStruct((1, 1), f32),
        ],
        scratch_shapes=[pltpu.SMEM((1,), f32)],
    )(et3, edge_attr, Wm_ef, bmsg.reshape(1, ED), tw2, tb2)

    p1 = pl.pallas_call(
        _node_pre_body,
        grid=(nbn,),
        in_specs=[
            pl.BlockSpec((BN, D), lambda i: (i, 0)),
            pl.BlockSpec((D, ED), lambda i: (0, 0)),
            pl.BlockSpec((1, ED), lambda i: (0, 0)),
        ],
        out_specs=pl.BlockSpec((BN, ED), lambda i: (i, 0)),
        out_shape=jax.ShapeDtypeStruct((N, ED), f32),
    )(node_features, W1x, b1.reshape(1, ED))

    g1 = _sc_gather(p1, src3)

    cnt_hist = pl.pallas_call(
        _hist_body,
        grid=(nbe,),
        in_specs=[pl.BlockSpec((1, 1, BE), lambda i: (i, 0, 0))],
        out_specs=pl.BlockSpec((N_PAD // 128, 128), lambda i: (0, 0)),
        out_shape=jax.ShapeDtypeStruct((N_PAD // 128, 128), f32),
    )(src_tc)
    cnt_tc = cnt_hist.reshape(N_PAD)[:N].reshape(nbn, 1, BN)

    msg = pl.pallas_call(
        _msg_body,
        grid=(nbe,),
        in_specs=[
            pl.BlockSpec((BE, ED), lambda i: (i, 0)),
            pl.BlockSpec((BE, DE), lambda i: (i, 0)),
            pl.BlockSpec((DE, ED), lambda i: (0, 0)),
            pl.BlockSpec((ED, MD), lambda i: (0, 0)),
            pl.BlockSpec((1, MD), lambda i: (0, 0)),
        ],
        out_specs=pl.BlockSpec((BE, MD), lambda i: (i, 0)),
        out_shape=jax.ShapeDtypeStruct((E, MD), f32),
    )(g1, edge_attr, W1e, W2, b2.reshape(1, MD))

    agg_p = _sc_scatter(msg, src3)

    paug, aout = pl.pallas_call(
        _gru_body,
        grid=(nbn,),
        in_specs=[
            pl.BlockSpec((2, BN, MD), lambda i: (0, i, 0)),
            pl.BlockSpec((1, 1, BN), lambda i: (i, 0, 0)),
            pl.BlockSpec((BN, D), lambda i: (i, 0)),
            pl.BlockSpec(memory_space=pltpu.SMEM),
            pl.BlockSpec((1, TD), lambda i: (0, 0)),
            pl.BlockSpec((1, TD), lambda i: (0, 0)),
            pl.BlockSpec((MD, 3 * MD), lambda i: (0, 0)),
            pl.BlockSpec((1, 3 * MD), lambda i: (0, 0)),
            pl.BlockSpec((1, 3 * MD), lambda i: (0, 0)),
            pl.BlockSpec((AUG, ED), lambda i: (0, 0)),
            pl.BlockSpec((AUG, ED), lambda i: (0, 0)),
            pl.BlockSpec((1, ED), lambda i: (0, 0)),
        ],
        out_specs=[
            pl.BlockSpec((BN, ED), lambda i: (i, 0)),
            pl.BlockSpec((BN, ED), lambda i: (i, 0)),
        ],
        out_shape=[
            jax.ShapeDtypeStruct((N, ED), f32),
            jax.ShapeDtypeStruct((N, ED), f32),
        ],
    )(agg_p, cnt_tc, node_features, latest, tw2, tb2, Wi, bi.reshape(1, 3 * MD),
      bh.reshape(1, 3 * MD), Wm_aug, Wo_aug, bout.reshape(1, ED))

    src4 = src.reshape(NW, KB // KC, KC, BB)
    dst4 = dst.reshape(NW, KB // KC, KC, BB)
    agg2_p = _sc_edgeb(paug, c2, src4, dst4)

    logits = pl.pallas_call(
        _final_body,
        grid=(nbn,),
        in_specs=[
            pl.BlockSpec((2, BN, ED), lambda i: (0, i, 0)),
            pl.BlockSpec((BN, ED), lambda i: (i, 0)),
            pl.BlockSpec((ED, ED), lambda i: (0, 0)),
            pl.BlockSpec((ED, ED // 2), lambda i: (0, 0)),
            pl.BlockSpec((1, ED // 2), lambda i: (0, 0)),
            pl.BlockSpec((ED // 2, 2), lambda i: (0, 0)),
            pl.BlockSpec((1, 2), lambda i: (0, 0)),
        ],
        out_specs=pl.BlockSpec((BN, 2), lambda i: (i, 0)),
        out_shape=jax.ShapeDtypeStruct((N, 2), f32),
    )(agg2_p, aout, Wo_agg, Wc1, bc1.reshape(1, ED // 2), Wc2,
      bc2.reshape(1, 2))

    return logits


# speedup vs baseline: 1.0446x; 1.0446x over previous
import functools

import jax
import jax.numpy as jnp
from jax import lax
from jax.experimental import pallas as pl
from jax.experimental.pallas import tpu as pltpu
from jax.experimental.pallas import tpu_sc as plsc

N = 10000
E = 320000
D = 128
DE = 16
TD = 32
MD = 128
ED = 128

NW = 32
EPW = E // NW
B = 80
K = EPW // B
BB = 40
KB = EPW // BB
KC = 50
N_PAD = 10240
RPT = N_PAD // 16

f32 = jnp.float32

_S = (0.9999997069578552, -0.16666577198040858, 0.008332557998290352,
      -0.00019812572236973137, 2.7040473311964398e-06,
      -2.053408005585339e-08)
_TWO_PI_HI = 6.28125
_TWO_PI_LO = 0.0019353071795864769


def _fast_sin(x):
    k = jnp.round(x * (1.0 / 6.283185307179586))
    r = (x - k * _TWO_PI_HI) - k * _TWO_PI_LO
    r2 = r * r
    s = _S[5]
    for c in (_S[4], _S[3], _S[2], _S[1], _S[0]):
        s = s * r2 + c
    return s * r


def _edge_pre_body(et_ref, ea_ref, src_ref, wef_ref, bmsg_ref, tw_ref, tb_ref,
                   c2_ref, lat_ref, cnt_ref, mx_ref):
    i = pl.program_id(0)
    et = et_ref[0, 0, :]
    te = _fast_sin(et[:, None] * tw_ref[0, :][None, :] + tb_ref[0, :][None, :])
    eafeat = jnp.concatenate([ea_ref[...], te], axis=1)
    c2_ref[...] = (jnp.dot(eafeat, wef_ref[...], preferred_element_type=f32)
                   + bmsg_ref[0, :][None, :])

    @pl.when(i == 0)
    def _():
        mx_ref[0] = -jnp.inf
        cnt_ref[...] = jnp.zeros_like(cnt_ref)

    mx_ref[0] = jnp.maximum(mx_ref[0], jnp.max(et))

    @pl.when(i == pl.num_programs(0) - 1)
    def _():
        lat_ref[0, 0] = mx_ref[0]

    srcv = src_ref[0, 0, :]
    g = (srcv[:, None] // 128
         == lax.broadcasted_iota(jnp.int32, (1, N_PAD // 128), 1)).astype(f32)
    o = (srcv[:, None] % 128
         == lax.broadcasted_iota(jnp.int32, (1, 128), 1)).astype(f32)
    cnt_ref[...] += lax.dot_general(g, o, (((0,), (0,)), ((), ())),
                                    preferred_element_type=f32)


def _node_pre_body(x_ref, w1x_ref, b1_ref, p1_ref):
    p1_ref[...] = (jnp.dot(x_ref[...], w1x_ref[...], preferred_element_type=f32)
                   + b1_ref[0, :][None, :])


def _msg_body(g1_ref, ea_ref, w1e_ref, w2_ref, b2_ref, out_ref):
    h = jnp.maximum(
        g1_ref[...] + jnp.dot(ea_ref[...], w1e_ref[...],
                              preferred_element_type=f32), 0.0)
    out_ref[...] = jnp.maximum(
        jnp.dot(h, w2_ref[...], preferred_element_type=f32)
        + b2_ref[0, :][None, :], 0.0)


def _gru_body(pa_ref, cnt_ref, x_ref, lat_ref, tw_ref, tb_ref, wi_ref, bi_ref,
              bh_ref, wma_ref, woa_ref, bout_ref, paug_ref, aout_ref):
    cnt = cnt_ref[0, 0, :]
    agg = (pa_ref[0] + pa_ref[1]) / jnp.maximum(cnt, 1.0)[:, None]
    gi = jnp.dot(agg, wi_ref[...], preferred_element_type=f32) + bi_ref[0][None, :]
    bh = bh_ref[0]
    r = jax.nn.sigmoid(gi[:, :MD] + bh[None, :MD])
    z = jax.nn.sigmoid(gi[:, MD:2 * MD] + bh[None, MD:2 * MD])
    nt = jnp.tanh(gi[:, 2 * MD:] + r * bh[None, 2 * MD:])
    mem = (1.0 - z) * nt
    gte = jnp.sin(lat_ref[0, 0] * tw_ref[0] + tb_ref[0])
    aug = jnp.concatenate(
        [x_ref[...], mem,
         jnp.broadcast_to(gte[None, :], (mem.shape[0], TD))], axis=1)
    paug_ref[...] = jnp.dot(aug, wma_ref[...], preferred_element_type=f32)
    aout_ref[...] = (jnp.dot(aug, woa_ref[...], preferred_element_type=f32)
                     + bout_ref[0][None, :])


def _final_body(pa_ref, aout_ref, wog_ref, wc1_ref, bc1_ref, wc2_ref, bc2_ref,
                out_ref):
    agg2 = pa_ref[0] + pa_ref[1]
    x = jnp.maximum(
        aout_ref[...] + jnp.dot(agg2, wog_ref[...], preferred_element_type=f32),
        0.0)
    h2 = jnp.maximum(
        jnp.dot(x, wc1_ref[...], preferred_element_type=f32)
        + bc1_ref[0][None, :], 0.0)
    out_ref[...] = (jnp.dot(h2, wc2_ref[...], preferred_element_type=f32)
                    + bc2_ref[0][None, :])


_MESH = plsc.VectorSubcoreMesh(core_axis_name="c", subcore_axis_name="s")


def _zero_accum(buf, accum, rs):
    nb = buf.shape[0]

    def zb(r, carry):
        for cc8 in range(8):
            buf[r, pl.ds(cc8 * 16, 16)] = jnp.zeros((16,), f32)
        return carry

    lax.fori_loop(0, nb, zb, 0)

    def cp(k, carry):
        off = pl.multiple_of(rs + k * nb, 8)
        pltpu.sync_copy(buf, accum.at[pl.ds(off, nb)])
        return carry

    lax.fori_loop(0, RPT // nb, cp, 0)


@functools.partial(
    pl.kernel, mesh=_MESH,
    out_type=jax.ShapeDtypeStruct((E, D), f32),
    scratch_types=[
        pltpu.VMEM((K, B), jnp.int32),
        pltpu.VMEM((B, D), f32),
        pltpu.VMEM((B, D), f32),
        pltpu.SemaphoreType.DMA,
        pltpu.SemaphoreType.DMA,
    ])
def _sc_gather(table_hbm, idx_hbm, out_hbm, idx_v, rows, rows1, sem, sem1):
    c = lax.axis_index("c")
    s = lax.axis_index("s")
    wid = s * 2 + c
    pltpu.sync_copy(idx_hbm.at[wid], idx_v)
    base = wid * EPW

    def body(t, carry):
        j0 = t * 2
        g0 = pltpu.make_async_copy(table_hbm.at[idx_v.at[j0]], rows, sem)
        g0.start()
        g1 = pltpu.make_async_copy(table_hbm.at[idx_v.at[j0 + 1]], rows1, sem1)
        g1.start()
        g0.wait()
        pltpu.sync_copy(rows, out_hbm.at[pl.ds(pl.multiple_of(base + j0 * B, 16), B)])
        g1.wait()
        pltpu.sync_copy(rows1, out_hbm.at[pl.ds(pl.multiple_of(base + (j0 + 1) * B, 16), B)])
        return carry

    lax.fori_loop(0, K // 2, body, 0)
    jt = K - 1
    pltpu.async_copy(table_hbm.at[idx_v.at[jt]], rows, sem).wait()
    pltpu.sync_copy(rows, out_hbm.at[pl.ds(pl.multiple_of(base + jt * B, 16), B)])


@functools.partial(
    pl.kernel, mesh=_MESH,
    out_type=jax.ShapeDtypeStruct((2, N_PAD, MD), f32),
    scratch_types=[
        pltpu.VMEM((K, B), jnp.int32),
        pltpu.VMEM((B, MD), f32),
        pltpu.VMEM((B, MD), f32),
        pltpu.VMEM_SHARED((N_PAD, MD), f32),
        pltpu.SemaphoreType.DMA,
        pltpu.SemaphoreType.DMA,
    ])
def _sc_scatter(msg_hbm, idx_hbm, out_hbm, idx_v, rows, rows1, accum, sem,
                sem1):
    c = lax.axis_index("c")
    s = lax.axis_index("s")
    wid = s * 2 + c
    rs = s * RPT
    _zero_accum(rows, accum, rs)
    plsc.subcore_barrier()
    pltpu.sync_copy(idx_hbm.at[wid], idx_v)
    base = wid * EPW

    def body(t, carry):
        j0 = t * 2
        off0 = pl.multiple_of(base + j0 * B, 16)
        off1 = pl.multiple_of(base + (j0 + 1) * B, 16)
        g0 = pltpu.make_async_copy(msg_hbm.at[pl.ds(off0, B)], rows, sem)
        g0.start()
        g1 = pltpu.make_async_copy(msg_hbm.at[pl.ds(off1, B)], rows1, sem1)
        g1.start()
        g0.wait()
        pltpu.sync_copy(rows, accum.at[idx_v.at[j0]], add=True)
        g1.wait()
        pltpu.sync_copy(rows1, accum.at[idx_v.at[j0 + 1]], add=True)
        return carry

    lax.fori_loop(0, K // 2, body, 0)
    jt = K - 1
    offt = pl.multiple_of(base + jt * B, 16)
    pltpu.async_copy(msg_hbm.at[pl.ds(offt, B)], rows, sem).wait()
    pltpu.sync_copy(rows, accum.at[idx_v.at[jt]], add=True)
    plsc.subcore_barrier()
    pltpu.sync_copy(accum.at[pl.ds(rs, RPT)], out_hbm.at[c, pl.ds(rs, RPT)])


@functools.partial(
    pl.kernel, mesh=_MESH,
    out_type=jax.ShapeDtypeStruct((2, N_PAD, ED), f32),
    scratch_types=[
        pltpu.VMEM((KC, BB), jnp.int32),
        pltpu.VMEM((KC, BB), jnp.int32),
        pltpu.VMEM((BB, ED), f32),
        pltpu.VMEM((BB, ED), f32),
        pltpu.VMEM((BB, ED), f32),
        pltpu.VMEM((BB, ED), f32),
        pltpu.VMEM_SHARED((N_PAD, ED), f32),
        pltpu.SemaphoreType.DMA,
        pltpu.SemaphoreType.DMA,
        pltpu.SemaphoreType.DMA,
        pltpu.SemaphoreType.DMA,
    ])
def _sc_edgeb(paug_hbm, c2_hbm, sidx_hbm, didx_hbm, out_hbm,
              sidx_v, didx_v, gbuf0, gbuf1, cbuf0, cbuf1, accum,
              gsem0, gsem1, csem0, csem1):
    c = lax.axis_index("c")
    s = lax.axis_index("s")
    wid = s * 2 + c
    rs = s * RPT
    _zero_accum(gbuf0, accum, rs)
    plsc.subcore_barrier()
    base = wid * EPW

    def _relu_add(gbuf, cbuf):
        def ew_row(r, carry):
            for cc8 in range(8):
                col = cc8 * 16
                v = gbuf[r, pl.ds(col, 16)] + cbuf[r, pl.ds(col, 16)]
                gbuf[r, pl.ds(col, 16)] = jnp.maximum(v, 0.0)
            return carry

        lax.fori_loop(0, BB, ew_row, 0)

    def outer(jj, carry):
        pltpu.sync_copy(sidx_hbm.at[wid, jj], sidx_v)
        pltpu.sync_copy(didx_hbm.at[wid, jj], didx_v)

        def body(t, carry2):
            j0 = t * 2
            off0 = pl.multiple_of(base + (jj * KC + j0) * BB, 8)
            off1 = pl.multiple_of(base + (jj * KC + j0 + 1) * BB, 8)
            c0 = pltpu.make_async_copy(c2_hbm.at[pl.ds(off0, BB)], cbuf0, csem0)
            c0.start()
            g0 = pltpu.make_async_copy(paug_hbm.at[sidx_v.at[j0]], gbuf0, gsem0)
            g0.start()
            c1 = pltpu.make_async_copy(c2_hbm.at[pl.ds(off1, BB)], cbuf1, csem1)
            c1.start()
            g1 = pltpu.make_async_copy(paug_hbm.at[sidx_v.at[j0 + 1]], gbuf1,
                                       gsem1)
            g1.start()
            g0.wait()
            c0.wait()
            _relu_add(gbuf0, cbuf0)
            pltpu.sync_copy(gbuf0, accum.at[didx_v.at[j0]], add=True)
            g1.wait()
            c1.wait()
            _relu_add(gbuf1, cbuf1)
            pltpu.sync_copy(gbuf1, accum.at[didx_v.at[j0 + 1]], add=True)
            return carry2

        lax.fori_loop(0, KC // 2, body, 0)
        return carry

    lax.fori_loop(0, KB // KC, outer, 0)
    plsc.subcore_barrier()
    pltpu.sync_copy(accum.at[pl.ds(rs, RPT)], out_hbm.at[c, pl.ds(rs, RPT)])


def kernel(node_features, edge_index, edge_attr, edge_times, time_w, time_b,
           W1, b1, W2, b2, Wi, bi, Wh, bh, Wmsg, bmsg, Wout, bout,
           Wc1, bc1, Wc2, bc2):
    del Wh
    BE = 2000
    BN = 2000
    nbe = E // BE
    nbn = N // BN
    AUG = D + MD + TD

    src = edge_index[0].astype(jnp.int32)
    dst = edge_index[1].astype(jnp.int32)
    src3 = src.reshape(NW, K, B)
    dst3 = dst.reshape(NW, K, B)
    src_tc = src.reshape(nbe, 1, BE)
    et3 = edge_times.reshape(nbe, 1, BE)
    tw2 = time_w.reshape(1, TD)
    tb2 = time_b.reshape(1, TD)

    W1x = W1[:D]
    W1e = W1[D + MD:]
    Wm_aug = Wmsg[:AUG]
    Wm_ef = Wmsg[AUG:]
    Wo_aug = Wout[:AUG]
    Wo_agg = Wout[AUG:]

    c2, latest, cnt_hist = pl.pallas_call(
        _edge_pre_body,
        grid=(nbe,),
        in_specs=[
            pl.BlockSpec((1, 1, BE), lambda i: (i, 0, 0)),
            pl.BlockSpec((BE, DE), lambda i: (i, 0)),
            pl.BlockSpec((1, 1, BE), lambda i: (i, 0, 0)),
            pl.BlockSpec((DE + TD, ED), lambda i: (0, 0)),
            pl.BlockSpec((1, ED), lambda i: (0, 0)),
            pl.BlockSpec((1, TD), lambda i: (0, 0)),
            pl.BlockSpec((1, TD), lambda i: (0, 0)),
        ],
        out_specs=[
            pl.BlockSpec((BE, ED), lambda i: (i, 0)),
            pl.BlockSpec(memory_space=pltpu.SMEM),
            pl.BlockSpec((N_PAD // 128, 128), lambda i: (0, 0)),
        ],
        out_shape=[
            jax.ShapeDtypeStruct((E, ED), f32),
            jax.ShapeDtypeStruct((1, 1), f32),
            jax.ShapeDtypeStruct((N_PAD // 128, 128), f32),
        ],
        scratch_shapes=[pltpu.SMEM((1,), f32)],
    )(et3, edge_attr, src_tc, Wm_ef, bmsg.reshape(1, ED), tw2, tb2)
    cnt_tc = cnt_hist.reshape(N_PAD)[:N].reshape(nbn, 1, BN)

    p1 = pl.pallas_call(
        _node_pre_body,
        grid=(nbn,),
        in_specs=[
            pl.BlockSpec((BN, D), lambda i: (i, 0)),
            pl.BlockSpec((D, ED), lambda i: (0, 0)),
            pl.BlockSpec((1, ED), lambda i: (0, 0)),
        ],
        out_specs=pl.BlockSpec((BN, ED), lambda i: (i, 0)),
        out_shape=jax.ShapeDtypeStruct((N, ED), f32),
    )(node_features, W1x, b1.reshape(1, ED))

    g1 = _sc_gather(p1, src3)

    msg = pl.pallas_call(
        _msg_body,
        grid=(nbe,),
        in_specs=[
            pl.BlockSpec((BE, ED), lambda i: (i, 0)),
            pl.BlockSpec((BE, DE), lambda i: (i, 0)),
            pl.BlockSpec((DE, ED), lambda i: (0, 0)),
            pl.BlockSpec((ED, MD), lambda i: (0, 0)),
            pl.BlockSpec((1, MD), lambda i: (0, 0)),
        ],
        out_specs=pl.BlockSpec((BE, MD), lambda i: (i, 0)),
        out_shape=jax.ShapeDtypeStruct((E, MD), f32),
    )(g1, edge_attr, W1e, W2, b2.reshape(1, MD))

    agg_p = _sc_scatter(msg, src3)

    paug, aout = pl.pallas_call(
        _gru_body,
        grid=(nbn,),
        in_specs=[
            pl.BlockSpec((2, BN, MD), lambda i: (0, i, 0)),
            pl.BlockSpec((1, 1, BN), lambda i: (i, 0, 0)),
            pl.BlockSpec((BN, D), lambda i: (i, 0)),
            pl.BlockSpec(memory_space=pltpu.SMEM),
            pl.BlockSpec((1, TD), lambda i: (0, 0)),
            pl.BlockSpec((1, TD), lambda i: (0, 0)),
            pl.BlockSpec((MD, 3 * MD), lambda i: (0, 0)),
            pl.BlockSpec((1, 3 * MD), lambda i: (0, 0)),
            pl.BlockSpec((1, 3 * MD), lambda i: (0, 0)),
            pl.BlockSpec((AUG, ED), lambda i: (0, 0)),
            pl.BlockSpec((AUG, ED), lambda i: (0, 0)),
            pl.BlockSpec((1, ED), lambda i: (0, 0)),
        ],
        out_specs=[
            pl.BlockSpec((BN, ED), lambda i: (i, 0)),
            pl.BlockSpec((BN, ED), lambda i: (i, 0)),
        ],
        out_shape=[
            jax.ShapeDtypeStruct((N, ED), f32),
            jax.ShapeDtypeStruct((N, ED), f32),
        ],
    )(agg_p, cnt_tc, node_features, latest, tw2, tb2, Wi, bi.reshape(1, 3 * MD),
      bh.reshape(1, 3 * MD), Wm_aug, Wo_aug, bout.reshape(1, ED))

    src4 = src.reshape(NW, KB // KC, KC, BB)
    dst4 = dst.reshape(NW, KB // KC, KC, BB)
    agg2_p = _sc_edgeb(paug, c2, src4, dst4)

    logits = pl.pallas_call(
        _final_body,
        grid=(nbn,),
        in_specs=[
            pl.BlockSpec((2, BN, ED), lambda i: (0, i, 0)),
            pl.BlockSpec((BN, ED), lambda i: (i, 0)),
            pl.BlockSpec((ED, ED), lambda i: (0, 0)),
            pl.BlockSpec((ED, ED // 2), lambda i: (0, 0)),
            pl.BlockSpec((1, ED // 2), lambda i: (0, 0)),
            pl.BlockSpec((ED // 2, 2), lambda i: (0, 0)),
            pl.BlockSpec((1, 2), lambda i: (0, 0)),
        ],
        out_specs=pl.BlockSpec((BN, 2), lambda i: (i, 0)),
        out_shape=jax.ShapeDtypeStruct((N, 2), f32),
    )(agg2_p, aout, Wo_agg, Wc1, bc1.reshape(1, ED // 2), Wc2,
      bc2.reshape(1, 2))

    return logits


# speedup vs baseline: 1.0492x; 1.0044x over previous
import functools

import jax
import jax.numpy as jnp
from jax import lax
from jax.experimental import pallas as pl
from jax.experimental.pallas import tpu as pltpu
from jax.experimental.pallas import tpu_sc as plsc

N = 10000
E = 320000
D = 128
DE = 16
TD = 32
MD = 128
ED = 128

NW = 32
EPW = E // NW
B = 80
K = EPW // B
BB = 40
KB = EPW // BB
KC = 50
N_PAD = 10240
RPT = N_PAD // 16

f32 = jnp.float32

_S = (0.9999997069578552, -0.16666577198040858, 0.008332557998290352,
      -0.00019812572236973137, 2.7040473311964398e-06,
      -2.053408005585339e-08)
_TWO_PI_HI = 6.28125
_TWO_PI_LO = 0.0019353071795864769


def _fast_sin(x):
    k = jnp.round(x * (1.0 / 6.283185307179586))
    r = (x - k * _TWO_PI_HI) - k * _TWO_PI_LO
    r2 = r * r
    s = _S[5]
    for c in (_S[4], _S[3], _S[2], _S[1], _S[0]):
        s = s * r2 + c
    return s * r


def _edge_pre_body(et_ref, ea_ref, src_ref, wef_ref, bmsg_ref, tw_ref, tb_ref,
                   c2_ref, lat_ref, cnt_ref, mx_ref):
    i = pl.program_id(0)
    et = et_ref[0, 0, :]
    te = _fast_sin(et[:, None] * tw_ref[0, :][None, :] + tb_ref[0, :][None, :])
    eafeat = jnp.concatenate([ea_ref[...], te], axis=1)
    c2_ref[...] = (jnp.dot(eafeat, wef_ref[...], preferred_element_type=f32)
                   + bmsg_ref[0, :][None, :])

    @pl.when(i == 0)
    def _():
        mx_ref[0] = -jnp.inf
        cnt_ref[...] = jnp.zeros_like(cnt_ref)

    mx_ref[0] = jnp.maximum(mx_ref[0], jnp.max(et))

    @pl.when(i == pl.num_programs(0) - 1)
    def _():
        lat_ref[0, 0] = mx_ref[0]

    srcv = src_ref[0, 0, :]
    g = (srcv[:, None] // 128
         == lax.broadcasted_iota(jnp.int32, (1, N_PAD // 128), 1)).astype(f32)
    o = (srcv[:, None] % 128
         == lax.broadcasted_iota(jnp.int32, (1, 128), 1)).astype(f32)
    cnt_ref[...] += lax.dot_general(g, o, (((0,), (0,)), ((), ())),
                                    preferred_element_type=f32)


def _node_pre_body(x_ref, w1x_ref, b1_ref, p1_ref):
    p1_ref[...] = (jnp.dot(x_ref[...], w1x_ref[...], preferred_element_type=f32)
                   + b1_ref[0, :][None, :])


def _msg_body(g1_ref, ea_ref, w1e_ref, w2_ref, b2_ref, _lat_ref, out_ref):
    h = jnp.maximum(
        g1_ref[...] + jnp.dot(ea_ref[...], w1e_ref[...],
                              preferred_element_type=f32), 0.0)
    out_ref[...] = jnp.maximum(
        jnp.dot(h, w2_ref[...], preferred_element_type=f32)
        + b2_ref[0, :][None, :], 0.0)


def _gru_body(pa_ref, cnt_ref, x_ref, lat_ref, tw_ref, tb_ref, wi_ref, bi_ref,
              bh_ref, wma_ref, woa_ref, bout_ref, paug_ref, aout_ref):
    cnt = cnt_ref[0, 0, :]
    agg = (pa_ref[0] + pa_ref[1]) / jnp.maximum(cnt, 1.0)[:, None]
    gi = jnp.dot(agg, wi_ref[...], preferred_element_type=f32) + bi_ref[0][None, :]
    bh = bh_ref[0]
    r = jax.nn.sigmoid(gi[:, :MD] + bh[None, :MD])
    z = jax.nn.sigmoid(gi[:, MD:2 * MD] + bh[None, MD:2 * MD])
    nt = jnp.tanh(gi[:, 2 * MD:] + r * bh[None, 2 * MD:])
    mem = (1.0 - z) * nt
    gte = jnp.sin(lat_ref[0, 0] * tw_ref[0] + tb_ref[0])
    aug = jnp.concatenate(
        [x_ref[...], mem,
         jnp.broadcast_to(gte[None, :], (mem.shape[0], TD))], axis=1)
    paug_ref[...] = jnp.dot(aug, wma_ref[...], preferred_element_type=f32)
    aout_ref[...] = (jnp.dot(aug, woa_ref[...], preferred_element_type=f32)
                     + bout_ref[0][None, :])


def _final_body(pa_ref, aout_ref, wog_ref, wc1_ref, bc1_ref, wc2_ref, bc2_ref,
                out_ref):
    agg2 = pa_ref[0] + pa_ref[1]
    x = jnp.maximum(
        aout_ref[...] + jnp.dot(agg2, wog_ref[...], preferred_element_type=f32),
        0.0)
    h2 = jnp.maximum(
        jnp.dot(x, wc1_ref[...], preferred_element_type=f32)
        + bc1_ref[0][None, :], 0.0)
    out_ref[...] = (jnp.dot(h2, wc2_ref[...], preferred_element_type=f32)
                    + bc2_ref[0][None, :])


_MESH = plsc.VectorSubcoreMesh(core_axis_name="c", subcore_axis_name="s")


def _zero_accum(buf, accum, rs):
    nb = buf.shape[0]

    def zb(r, carry):
        for cc8 in range(8):
            buf[r, pl.ds(cc8 * 16, 16)] = jnp.zeros((16,), f32)
        return carry

    lax.fori_loop(0, nb, zb, 0)

    def cp(k, carry):
        off = pl.multiple_of(rs + k * nb, 8)
        pltpu.sync_copy(buf, accum.at[pl.ds(off, nb)])
        return carry

    lax.fori_loop(0, RPT // nb, cp, 0)


@functools.partial(
    pl.kernel, mesh=_MESH,
    out_type=jax.ShapeDtypeStruct((E, D), f32),
    scratch_types=[
        pltpu.VMEM((K, B), jnp.int32),
        pltpu.VMEM((B, D), f32),
        pltpu.VMEM((B, D), f32),
        pltpu.SemaphoreType.DMA,
        pltpu.SemaphoreType.DMA,
    ])
def _sc_gather(table_hbm, idx_hbm, out_hbm, idx_v, rows, rows1, sem, sem1):
    c = lax.axis_index("c")
    s = lax.axis_index("s")
    wid = s * 2 + c
    pltpu.sync_copy(idx_hbm.at[wid], idx_v)
    base = wid * EPW

    def body(t, carry):
        j0 = t * 2
        g0 = pltpu.make_async_copy(table_hbm.at[idx_v.at[j0]], rows, sem)
        g0.start()
        g1 = pltpu.make_async_copy(table_hbm.at[idx_v.at[j0 + 1]], rows1, sem1)
        g1.start()
        g0.wait()
        pltpu.sync_copy(rows, out_hbm.at[pl.ds(pl.multiple_of(base + j0 * B, 16), B)])
        g1.wait()
        pltpu.sync_copy(rows1, out_hbm.at[pl.ds(pl.multiple_of(base + (j0 + 1) * B, 16), B)])
        return carry

    lax.fori_loop(0, K // 2, body, 0)
    jt = K - 1
    pltpu.async_copy(table_hbm.at[idx_v.at[jt]], rows, sem).wait()
    pltpu.sync_copy(rows, out_hbm.at[pl.ds(pl.multiple_of(base + jt * B, 16), B)])


@functools.partial(
    pl.kernel, mesh=_MESH,
    out_type=jax.ShapeDtypeStruct((2, N_PAD, MD), f32),
    scratch_types=[
        pltpu.VMEM((K, B), jnp.int32),
        pltpu.VMEM((B, MD), f32),
        pltpu.VMEM((B, MD), f32),
        pltpu.VMEM_SHARED((N_PAD, MD), f32),
        pltpu.SemaphoreType.DMA,
        pltpu.SemaphoreType.DMA,
    ])
def _sc_scatter(msg_hbm, idx_hbm, out_hbm, idx_v, rows, rows1, accum, sem,
                sem1):
    c = lax.axis_index("c")
    s = lax.axis_index("s")
    wid = s * 2 + c
    rs = s * RPT
    _zero_accum(rows, accum, rs)
    plsc.subcore_barrier()
    pltpu.sync_copy(idx_hbm.at[wid], idx_v)
    base = wid * EPW

    def body(t, carry):
        j0 = t * 2
        off0 = pl.multiple_of(base + j0 * B, 16)
        off1 = pl.multiple_of(base + (j0 + 1) * B, 16)
        g0 = pltpu.make_async_copy(msg_hbm.at[pl.ds(off0, B)], rows, sem)
        g0.start()
        g1 = pltpu.make_async_copy(msg_hbm.at[pl.ds(off1, B)], rows1, sem1)
        g1.start()
        g0.wait()
        pltpu.sync_copy(rows, accum.at[idx_v.at[j0]], add=True)
        g1.wait()
        pltpu.sync_copy(rows1, accum.at[idx_v.at[j0 + 1]], add=True)
        return carry

    lax.fori_loop(0, K // 2, body, 0)
    jt = K - 1
    offt = pl.multiple_of(base + jt * B, 16)
    pltpu.async_copy(msg_hbm.at[pl.ds(offt, B)], rows, sem).wait()
    pltpu.sync_copy(rows, accum.at[idx_v.at[jt]], add=True)
    plsc.subcore_barrier()
    pltpu.sync_copy(accum.at[pl.ds(rs, RPT)], out_hbm.at[c, pl.ds(rs, RPT)])


@functools.partial(
    pl.kernel, mesh=_MESH,
    out_type=jax.ShapeDtypeStruct((2, N_PAD, ED), f32),
    scratch_types=[
        pltpu.VMEM((KC, BB), jnp.int32),
        pltpu.VMEM((KC, BB), jnp.int32),
        pltpu.VMEM((BB, ED), f32),
        pltpu.VMEM((BB, ED), f32),
        pltpu.VMEM((BB, ED), f32),
        pltpu.VMEM((BB, ED), f32),
        pltpu.VMEM_SHARED((N_PAD, ED), f32),
        pltpu.SemaphoreType.DMA,
        pltpu.SemaphoreType.DMA,
        pltpu.SemaphoreType.DMA,
        pltpu.SemaphoreType.DMA,
    ])
def _sc_edgeb(paug_hbm, c2a_hbm, c2b_hbm, sidx_hbm, didx_hbm, out_hbm,
              sidx_v, didx_v, gbuf0, gbuf1, cbuf0, cbuf1, accum,
              gsem0, gsem1, csem0, csem1):
    c = lax.axis_index("c")
    s = lax.axis_index("s")
    wid = s * 2 + c
    rs = s * RPT
    _zero_accum(gbuf0, accum, rs)
    plsc.subcore_barrier()
    base = wid * EPW

    def _relu_add(gbuf, cbuf):
        def ew_row(r, carry):
            for cc8 in range(8):
                col = cc8 * 16
                v = gbuf[r, pl.ds(col, 16)] + cbuf[r, pl.ds(col, 16)]
                gbuf[r, pl.ds(col, 16)] = jnp.maximum(v, 0.0)
            return carry

        lax.fori_loop(0, BB, ew_row, 0)

    def _run(c2_hbm, cbase):
        def outer(jj, carry):
            pltpu.sync_copy(sidx_hbm.at[wid, jj], sidx_v)
            pltpu.sync_copy(didx_hbm.at[wid, jj], didx_v)

            def body(t, carry2):
                j0 = t * 2
                off0 = pl.multiple_of(cbase + (jj * KC + j0) * BB, 8)
                off1 = pl.multiple_of(cbase + (jj * KC + j0 + 1) * BB, 8)
                c0 = pltpu.make_async_copy(c2_hbm.at[pl.ds(off0, BB)], cbuf0,
                                           csem0)
                c0.start()
                g0 = pltpu.make_async_copy(paug_hbm.at[sidx_v.at[j0]], gbuf0,
                                           gsem0)
                g0.start()
                c1 = pltpu.make_async_copy(c2_hbm.at[pl.ds(off1, BB)], cbuf1,
                                           csem1)
                c1.start()
                g1 = pltpu.make_async_copy(paug_hbm.at[sidx_v.at[j0 + 1]],
                                           gbuf1, gsem1)
                g1.start()
                g0.wait()
                c0.wait()
                _relu_add(gbuf0, cbuf0)
                pltpu.sync_copy(gbuf0, accum.at[didx_v.at[j0]], add=True)
                g1.wait()
                c1.wait()
                _relu_add(gbuf1, cbuf1)
                pltpu.sync_copy(gbuf1, accum.at[didx_v.at[j0 + 1]], add=True)
                return carry2

            lax.fori_loop(0, KC // 2, body, 0)
            return carry

        lax.fori_loop(0, KB // KC, outer, 0)

    @pl.when(wid < NW // 2)
    def _():
        _run(c2a_hbm, base)

    @pl.when(wid >= NW // 2)
    def _():
        _run(c2b_hbm, base - (E // 2))

    plsc.subcore_barrier()
    pltpu.sync_copy(accum.at[pl.ds(rs, RPT)], out_hbm.at[c, pl.ds(rs, RPT)])


def kernel(node_features, edge_index, edge_attr, edge_times, time_w, time_b,
           W1, b1, W2, b2, Wi, bi, Wh, bh, Wmsg, bmsg, Wout, bout,
           Wc1, bc1, Wc2, bc2):
    del Wh
    BE = 2000
    BN = 2000
    nbe = E // BE
    nbn = N // BN
    AUG = D + MD + TD

    src = edge_index[0].astype(jnp.int32)
    dst = edge_index[1].astype(jnp.int32)
    src3 = src.reshape(NW, K, B)
    dst3 = dst.reshape(NW, K, B)
    src_tc = src.reshape(nbe, 1, BE)
    et3 = edge_times.reshape(nbe, 1, BE)
    tw2 = time_w.reshape(1, TD)
    tb2 = time_b.reshape(1, TD)

    W1x = W1[:D]
    W1e = W1[D + MD:]
    Wm_aug = Wmsg[:AUG]
    Wm_ef = Wmsg[AUG:]
    Wo_aug = Wout[:AUG]
    Wo_agg = Wout[AUG:]

    nbe2 = nbe // 2

    def _edge_pre_call(q):
        return pl.pallas_call(
            _edge_pre_body,
            grid=(nbe2,),
            in_specs=[
                pl.BlockSpec((1, 1, BE), lambda i, q=q: (i + q * nbe2, 0, 0)),
                pl.BlockSpec((BE, DE), lambda i, q=q: (i + q * nbe2, 0)),
                pl.BlockSpec((1, 1, BE), lambda i, q=q: (i + q * nbe2, 0, 0)),
                pl.BlockSpec((DE + TD, ED), lambda i: (0, 0)),
                pl.BlockSpec((1, ED), lambda i: (0, 0)),
                pl.BlockSpec((1, TD), lambda i: (0, 0)),
                pl.BlockSpec((1, TD), lambda i: (0, 0)),
            ],
            out_specs=[
                pl.BlockSpec((BE, ED), lambda i: (i, 0)),
                pl.BlockSpec(memory_space=pltpu.SMEM),
                pl.BlockSpec((N_PAD // 128, 128), lambda i: (0, 0)),
            ],
            out_shape=[
                jax.ShapeDtypeStruct((E // 2, ED), f32),
                jax.ShapeDtypeStruct((1, 1), f32),
                jax.ShapeDtypeStruct((N_PAD // 128, 128), f32),
            ],
            scratch_shapes=[pltpu.SMEM((1,), f32)],
        )(et3, edge_attr, src_tc, Wm_ef, bmsg.reshape(1, ED), tw2, tb2)

    c2a, lat_a, cnt_a = _edge_pre_call(0)
    c2b, lat_b, cnt_b = _edge_pre_call(1)
    latest = jnp.maximum(lat_a, lat_b)
    cnt_tc = (cnt_a + cnt_b).reshape(N_PAD)[:N].reshape(nbn, 1, BN)

    p1 = pl.pallas_call(
        _node_pre_body,
        grid=(nbn,),
        in_specs=[
            pl.BlockSpec((BN, D), lambda i: (i, 0)),
            pl.BlockSpec((D, ED), lambda i: (0, 0)),
            pl.BlockSpec((1, ED), lambda i: (0, 0)),
        ],
        out_specs=pl.BlockSpec((BN, ED), lambda i: (i, 0)),
        out_shape=jax.ShapeDtypeStruct((N, ED), f32),
    )(node_features, W1x, b1.reshape(1, ED))

    g1 = _sc_gather(p1, src3)

    msg = pl.pallas_call(
        _msg_body,
        grid=(nbe,),
        in_specs=[
            pl.BlockSpec((BE, ED), lambda i: (i, 0)),
            pl.BlockSpec((BE, DE), lambda i: (i, 0)),
            pl.BlockSpec((DE, ED), lambda i: (0, 0)),
            pl.BlockSpec((ED, MD), lambda i: (0, 0)),
            pl.BlockSpec((1, MD), lambda i: (0, 0)),
            pl.BlockSpec(memory_space=pltpu.SMEM),
        ],
        out_specs=pl.BlockSpec((BE, MD), lambda i: (i, 0)),
        out_shape=jax.ShapeDtypeStruct((E, MD), f32),
    )(g1, edge_attr, W1e, W2, b2.reshape(1, MD), lat_a)

    agg_p = _sc_scatter(msg, src3)

    paug, aout = pl.pallas_call(
        _gru_body,
        grid=(nbn,),
        in_specs=[
            pl.BlockSpec((2, BN, MD), lambda i: (0, i, 0)),
            pl.BlockSpec((1, 1, BN), lambda i: (i, 0, 0)),
            pl.BlockSpec((BN, D), lambda i: (i, 0)),
            pl.BlockSpec(memory_space=pltpu.SMEM),
            pl.BlockSpec((1, TD), lambda i: (0, 0)),
            pl.BlockSpec((1, TD), lambda i: (0, 0)),
            pl.BlockSpec((MD, 3 * MD), lambda i: (0, 0)),
            pl.BlockSpec((1, 3 * MD), lambda i: (0, 0)),
            pl.BlockSpec((1, 3 * MD), lambda i: (0, 0)),
            pl.BlockSpec((AUG, ED), lambda i: (0, 0)),
            pl.BlockSpec((AUG, ED), lambda i: (0, 0)),
            pl.BlockSpec((1, ED), lambda i: (0, 0)),
        ],
        out_specs=[
            pl.BlockSpec((BN, ED), lambda i: (i, 0)),
            pl.BlockSpec((BN, ED), lambda i: (i, 0)),
        ],
        out_shape=[
            jax.ShapeDtypeStruct((N, ED), f32),
            jax.ShapeDtypeStruct((N, ED), f32),
        ],
    )(agg_p, cnt_tc, node_features, latest, tw2, tb2, Wi, bi.reshape(1, 3 * MD),
      bh.reshape(1, 3 * MD), Wm_aug, Wo_aug, bout.reshape(1, ED))

    src4 = src.reshape(NW, KB // KC, KC, BB)
    dst4 = dst.reshape(NW, KB // KC, KC, BB)
    agg2_p = _sc_edgeb(paug, c2a, c2b, src4, dst4)

    logits = pl.pallas_call(
        _final_body,
        grid=(nbn,),
        in_specs=[
            pl.BlockSpec((2, BN, ED), lambda i: (0, i, 0)),
            pl.BlockSpec((BN, ED), lambda i: (i, 0)),
            pl.BlockSpec((ED, ED), lambda i: (0, 0)),
            pl.BlockSpec((ED, ED // 2), lambda i: (0, 0)),
            pl.BlockSpec((1, ED // 2), lambda i: (0, 0)),
            pl.BlockSpec((ED // 2, 2), lambda i: (0, 0)),
            pl.BlockSpec((1, 2), lambda i: (0, 0)),
        ],
        out_specs=pl.BlockSpec((BN, 2), lambda i: (i, 0)),
        out_shape=jax.ShapeDtypeStruct((N, 2), f32),
    )(agg2_p, aout, Wo_agg, Wc1, bc1.reshape(1, ED // 2), Wc2,
      bc2.reshape(1, 2))

    return logits


# speedup vs baseline: 1.1150x; 1.0627x over previous
import functools

import jax
import jax.numpy as jnp
from jax import lax
from jax.experimental import pallas as pl
from jax.experimental.pallas import tpu as pltpu
from jax.experimental.pallas import tpu_sc as plsc

N = 10000
E = 320000
D = 128
DE = 16
TD = 32
MD = 128
ED = 128

NW = 32
EPW = E // NW
B = 80
K = EPW // B
BB = 40
KB = EPW // BB
KC = 50
N_PAD = 10240
RPT = N_PAD // 16

f32 = jnp.float32

_S = (0.9999997069578552, -0.16666577198040858, 0.008332557998290352,
      -0.00019812572236973137, 2.7040473311964398e-06,
      -2.053408005585339e-08)
_TWO_PI_HI = 6.28125
_TWO_PI_LO = 0.0019353071795864769


def _fast_sin(x):
    k = jnp.round(x * (1.0 / 6.283185307179586))
    r = (x - k * _TWO_PI_HI) - k * _TWO_PI_LO
    r2 = r * r
    s = _S[5]
    for c in (_S[4], _S[3], _S[2], _S[1], _S[0]):
        s = s * r2 + c
    return s * r


def _edge_pre_body(et_ref, ea_ref, wef_ref, bmsg_ref, tw_ref, tb_ref,
                   c2_ref, lat_ref, mx_ref):
    i = pl.program_id(0)
    et = et_ref[0, 0, :]
    te = _fast_sin(et[:, None] * tw_ref[0, :][None, :] + tb_ref[0, :][None, :])
    eafeat = jnp.concatenate([ea_ref[...], te], axis=1)
    c2_ref[...] = (jnp.dot(eafeat, wef_ref[...], preferred_element_type=f32)
                   + bmsg_ref[0, :][None, :])

    @pl.when(i == 0)
    def _():
        mx_ref[0] = -jnp.inf

    mx_ref[0] = jnp.maximum(mx_ref[0], jnp.max(et))

    @pl.when(i == pl.num_programs(0) - 1)
    def _():
        lat_ref[0, 0] = mx_ref[0]


def _node_pre_body(x_ref, w1x_ref, b1_ref, p1_ref):
    p1_ref[...] = (jnp.dot(x_ref[...], w1x_ref[...], preferred_element_type=f32)
                   + b1_ref[0, :][None, :])


def _msg_body(g1_ref, ea_ref, src_ref, w1e_ref, w2_ref, b2_ref, out_ref,
              cnt_ref):
    i = pl.program_id(0)
    h = jnp.maximum(
        g1_ref[...] + jnp.dot(ea_ref[...], w1e_ref[...],
                              preferred_element_type=f32), 0.0)
    out_ref[...] = jnp.maximum(
        jnp.dot(h, w2_ref[...], preferred_element_type=f32)
        + b2_ref[0, :][None, :], 0.0)
    srcv = src_ref[0, 0, :]
    g = (srcv[:, None] // 128
         == lax.broadcasted_iota(jnp.int32, (1, N_PAD // 128), 1)).astype(f32)
    o = (srcv[:, None] % 128
         == lax.broadcasted_iota(jnp.int32, (1, 128), 1)).astype(f32)
    contrib = lax.dot_general(g, o, (((0,), (0,)), ((), ())),
                              preferred_element_type=f32)

    @pl.when(i == 0)
    def _():
        cnt_ref[...] = jnp.zeros_like(cnt_ref)

    cnt_ref[...] += contrib


def _gru_body(pa_ref, cnt_ref, x_ref, lat_ref, tw_ref, tb_ref, wi_ref, bi_ref,
              bh_ref, wma_ref, woa_ref, bout_ref, paug_ref, aout_ref):
    cnt = cnt_ref[0, 0, :]
    agg = (pa_ref[0] + pa_ref[1]) / jnp.maximum(cnt, 1.0)[:, None]
    gi = jnp.dot(agg, wi_ref[...], preferred_element_type=f32) + bi_ref[0][None, :]
    bh = bh_ref[0]
    r = jax.nn.sigmoid(gi[:, :MD] + bh[None, :MD])
    z = jax.nn.sigmoid(gi[:, MD:2 * MD] + bh[None, MD:2 * MD])
    nt = jnp.tanh(gi[:, 2 * MD:] + r * bh[None, 2 * MD:])
    mem = (1.0 - z) * nt
    gte = jnp.sin(lat_ref[0, 0] * tw_ref[0] + tb_ref[0])
    aug = jnp.concatenate(
        [x_ref[...], mem,
         jnp.broadcast_to(gte[None, :], (mem.shape[0], TD))], axis=1)
    paug_ref[...] = jnp.dot(aug, wma_ref[...], preferred_element_type=f32)
    aout_ref[...] = (jnp.dot(aug, woa_ref[...], preferred_element_type=f32)
                     + bout_ref[0][None, :])


def _final_body(pa_ref, aout_ref, wog_ref, wc1_ref, bc1_ref, wc2_ref, bc2_ref,
                out_ref):
    agg2 = pa_ref[0] + pa_ref[1]
    x = jnp.maximum(
        aout_ref[...] + jnp.dot(agg2, wog_ref[...], preferred_element_type=f32),
        0.0)
    h2 = jnp.maximum(
        jnp.dot(x, wc1_ref[...], preferred_element_type=f32)
        + bc1_ref[0][None, :], 0.0)
    out_ref[...] = (jnp.dot(h2, wc2_ref[...], preferred_element_type=f32)
                    + bc2_ref[0][None, :])


_MESH = plsc.VectorSubcoreMesh(core_axis_name="c", subcore_axis_name="s")


def _zero_accum(buf, accum, rs):
    nb = buf.shape[0]

    def zb(r, carry):
        for cc8 in range(8):
            buf[r, pl.ds(cc8 * 16, 16)] = jnp.zeros((16,), f32)
        return carry

    lax.fori_loop(0, nb, zb, 0)

    def cp(k, carry):
        off = pl.multiple_of(rs + k * nb, 8)
        pltpu.sync_copy(buf, accum.at[pl.ds(off, nb)])
        return carry

    lax.fori_loop(0, RPT // nb, cp, 0)


@functools.partial(
    pl.kernel, mesh=_MESH,
    out_type=jax.ShapeDtypeStruct((E, D), f32),
    scratch_types=[
        pltpu.VMEM((K, B), jnp.int32),
        pltpu.VMEM((B, D), f32),
        pltpu.VMEM((B, D), f32),
        pltpu.SemaphoreType.DMA,
        pltpu.SemaphoreType.DMA,
    ])
def _sc_gather(table_hbm, idx_hbm, out_hbm, idx_v, rows, rows1, sem, sem1):
    c = lax.axis_index("c")
    s = lax.axis_index("s")
    wid = s * 2 + c
    pltpu.sync_copy(idx_hbm.at[wid], idx_v)
    base = wid * EPW

    def body(t, carry):
        j0 = t * 2
        g0 = pltpu.make_async_copy(table_hbm.at[idx_v.at[j0]], rows, sem)
        g0.start()
        g1 = pltpu.make_async_copy(table_hbm.at[idx_v.at[j0 + 1]], rows1, sem1)
        g1.start()
        g0.wait()
        pltpu.sync_copy(rows, out_hbm.at[pl.ds(pl.multiple_of(base + j0 * B, 16), B)])
        g1.wait()
        pltpu.sync_copy(rows1, out_hbm.at[pl.ds(pl.multiple_of(base + (j0 + 1) * B, 16), B)])
        return carry

    lax.fori_loop(0, K // 2, body, 0)
    jt = K - 1
    pltpu.async_copy(table_hbm.at[idx_v.at[jt]], rows, sem).wait()
    pltpu.sync_copy(rows, out_hbm.at[pl.ds(pl.multiple_of(base + jt * B, 16), B)])


@functools.partial(
    pl.kernel, mesh=_MESH,
    out_type=jax.ShapeDtypeStruct((2, N_PAD, MD), f32),
    scratch_types=[
        pltpu.VMEM((K, B), jnp.int32),
        pltpu.VMEM((B, MD), f32),
        pltpu.VMEM((B, MD), f32),
        pltpu.VMEM_SHARED((N_PAD, MD), f32),
        pltpu.SemaphoreType.DMA,
        pltpu.SemaphoreType.DMA,
    ])
def _sc_scatter(msg_hbm, idx_hbm, out_hbm, idx_v, rows, rows1, accum, sem,
                sem1):
    c = lax.axis_index("c")
    s = lax.axis_index("s")
    wid = s * 2 + c
    rs = s * RPT
    _zero_accum(rows, accum, rs)
    plsc.subcore_barrier()
    pltpu.sync_copy(idx_hbm.at[wid], idx_v)
    base = wid * EPW

    def body(t, carry):
        j0 = t * 2
        off0 = pl.multiple_of(base + j0 * B, 16)
        off1 = pl.multiple_of(base + (j0 + 1) * B, 16)
        g0 = pltpu.make_async_copy(msg_hbm.at[pl.ds(off0, B)], rows, sem)
        g0.start()
        g1 = pltpu.make_async_copy(msg_hbm.at[pl.ds(off1, B)], rows1, sem1)
        g1.start()
        g0.wait()
        pltpu.sync_copy(rows, accum.at[idx_v.at[j0]], add=True)
        g1.wait()
        pltpu.sync_copy(rows1, accum.at[idx_v.at[j0 + 1]], add=True)
        return carry

    lax.fori_loop(0, K // 2, body, 0)
    jt = K - 1
    offt = pl.multiple_of(base + jt * B, 16)
    pltpu.async_copy(msg_hbm.at[pl.ds(offt, B)], rows, sem).wait()
    pltpu.sync_copy(rows, accum.at[idx_v.at[jt]], add=True)
    plsc.subcore_barrier()
    pltpu.sync_copy(accum.at[pl.ds(rs, RPT)], out_hbm.at[c, pl.ds(rs, RPT)])


@functools.partial(
    pl.kernel, mesh=_MESH,
    out_type=jax.ShapeDtypeStruct((2, N_PAD, ED), f32),
    scratch_types=[
        pltpu.VMEM((KC, BB), jnp.int32),
        pltpu.VMEM((KC, BB), jnp.int32),
        pltpu.VMEM((BB, ED), f32),
        pltpu.VMEM((BB, ED), f32),
        pltpu.VMEM((BB, ED), f32),
        pltpu.VMEM((BB, ED), f32),
        pltpu.VMEM_SHARED((N_PAD, ED), f32),
        pltpu.SemaphoreType.DMA,
        pltpu.SemaphoreType.DMA,
        pltpu.SemaphoreType.DMA,
        pltpu.SemaphoreType.DMA,
    ])
def _sc_edgeb(paug_hbm, c2_hbm, sidx_hbm, didx_hbm, out_hbm,
              sidx_v, didx_v, gbuf0, gbuf1, cbuf0, cbuf1, accum,
              gsem0, gsem1, csem0, csem1):
    c = lax.axis_index("c")
    s = lax.axis_index("s")
    wid = s * 2 + c
    rs = s * RPT
    _zero_accum(gbuf0, accum, rs)
    plsc.subcore_barrier()
    base = wid * EPW

    def _relu_add(gbuf, cbuf):
        def ew_row(r, carry):
            for cc8 in range(8):
                col = cc8 * 16
                v = gbuf[r, pl.ds(col, 16)] + cbuf[r, pl.ds(col, 16)]
                gbuf[r, pl.ds(col, 16)] = jnp.maximum(v, 0.0)
            return carry

        lax.fori_loop(0, BB, ew_row, 0)

    def _run(c2_hbm, cbase):
        def outer(jj, carry):
            pltpu.sync_copy(sidx_hbm.at[wid, jj], sidx_v)
            pltpu.sync_copy(didx_hbm.at[wid, jj], didx_v)

            def body(t, carry2):
                j0 = t * 2
                off0 = pl.multiple_of(cbase + (jj * KC + j0) * BB, 8)
                off1 = pl.multiple_of(cbase + (jj * KC + j0 + 1) * BB, 8)
                c0 = pltpu.make_async_copy(c2_hbm.at[pl.ds(off0, BB)], cbuf0,
                                           csem0)
                c0.start()
                g0 = pltpu.make_async_copy(paug_hbm.at[sidx_v.at[j0]], gbuf0,
                                           gsem0)
                g0.start()
                c1 = pltpu.make_async_copy(c2_hbm.at[pl.ds(off1, BB)], cbuf1,
                                           csem1)
                c1.start()
                g1 = pltpu.make_async_copy(paug_hbm.at[sidx_v.at[j0 + 1]],
                                           gbuf1, gsem1)
                g1.start()
                g0.wait()
                c0.wait()
                _relu_add(gbuf0, cbuf0)
                pltpu.sync_copy(gbuf0, accum.at[didx_v.at[j0]], add=True)
                g1.wait()
                c1.wait()
                _relu_add(gbuf1, cbuf1)
                pltpu.sync_copy(gbuf1, accum.at[didx_v.at[j0 + 1]], add=True)
                return carry2

            lax.fori_loop(0, KC // 2, body, 0)
            return carry

        lax.fori_loop(0, KB // KC, outer, 0)

    _run(c2_hbm, base)
    plsc.subcore_barrier()
    pltpu.sync_copy(accum.at[pl.ds(rs, RPT)], out_hbm.at[c, pl.ds(rs, RPT)])


def kernel(node_features, edge_index, edge_attr, edge_times, time_w, time_b,
           W1, b1, W2, b2, Wi, bi, Wh, bh, Wmsg, bmsg, Wout, bout,
           Wc1, bc1, Wc2, bc2):
    del Wh
    BE = 2000
    BN = 2000
    nbe = E // BE
    nbn = N // BN
    AUG = D + MD + TD

    src = edge_index[0].astype(jnp.int32)
    dst = edge_index[1].astype(jnp.int32)
    src3 = src.reshape(NW, K, B)
    dst3 = dst.reshape(NW, K, B)
    src_tc = src.reshape(nbe, 1, BE)
    et3 = edge_times.reshape(nbe, 1, BE)
    tw2 = time_w.reshape(1, TD)
    tb2 = time_b.reshape(1, TD)

    W1x = W1[:D]
    W1e = W1[D + MD:]
    Wm_aug = Wmsg[:AUG]
    Wm_ef = Wmsg[AUG:]
    Wo_aug = Wout[:AUG]
    Wo_agg = Wout[AUG:]

    c2, latest = pl.pallas_call(
        _edge_pre_body,
        grid=(nbe,),
        in_specs=[
            pl.BlockSpec((1, 1, BE), lambda i: (i, 0, 0)),
            pl.BlockSpec((BE, DE), lambda i: (i, 0)),
            pl.BlockSpec((DE + TD, ED), lambda i: (0, 0)),
            pl.BlockSpec((1, ED), lambda i: (0, 0)),
            pl.BlockSpec((1, TD), lambda i: (0, 0)),
            pl.BlockSpec((1, TD), lambda i: (0, 0)),
        ],
        out_specs=[
            pl.BlockSpec((BE, ED), lambda i: (i, 0)),
            pl.BlockSpec(memory_space=pltpu.SMEM),
        ],
        out_shape=[
            jax.ShapeDtypeStruct((E, ED), f32),
            jax.ShapeDtypeStruct((1, 1), f32),
        ],
        scratch_shapes=[pltpu.SMEM((1,), f32)],
    )(et3, edge_attr, Wm_ef, bmsg.reshape(1, ED), tw2, tb2)

    p1 = pl.pallas_call(
        _node_pre_body,
        grid=(nbn,),
        in_specs=[
            pl.BlockSpec((BN, D), lambda i: (i, 0)),
            pl.BlockSpec((D, ED), lambda i: (0, 0)),
            pl.BlockSpec((1, ED), lambda i: (0, 0)),
        ],
        out_specs=pl.BlockSpec((BN, ED), lambda i: (i, 0)),
        out_shape=jax.ShapeDtypeStruct((N, ED), f32),
    )(node_features, W1x, b1.reshape(1, ED))

    g1 = _sc_gather(p1, src3)

    msg, cnt_hist = pl.pallas_call(
        _msg_body,
        grid=(nbe,),
        in_specs=[
            pl.BlockSpec((BE, ED), lambda i: (i, 0)),
            pl.BlockSpec((BE, DE), lambda i: (i, 0)),
            pl.BlockSpec((1, 1, BE), lambda i: (i, 0, 0)),
            pl.BlockSpec((DE, ED), lambda i: (0, 0)),
            pl.BlockSpec((ED, MD), lambda i: (0, 0)),
            pl.BlockSpec((1, MD), lambda i: (0, 0)),
        ],
        out_specs=[
            pl.BlockSpec((BE, MD), lambda i: (i, 0)),
            pl.BlockSpec((N_PAD // 128, 128), lambda i: (0, 0)),
        ],
        out_shape=[
            jax.ShapeDtypeStruct((E, MD), f32),
            jax.ShapeDtypeStruct((N_PAD // 128, 128), f32),
        ],
    )(g1, edge_attr, src_tc, W1e, W2, b2.reshape(1, MD))
    cnt_tc = cnt_hist.reshape(N_PAD)[:N].reshape(nbn, 1, BN)

    agg_p = _sc_scatter(msg, src3)

    paug, aout = pl.pallas_call(
        _gru_body,
        grid=(nbn,),
        in_specs=[
            pl.BlockSpec((2, BN, MD), lambda i: (0, i, 0)),
            pl.BlockSpec((1, 1, BN), lambda i: (i, 0, 0)),
            pl.BlockSpec((BN, D), lambda i: (i, 0)),
            pl.BlockSpec(memory_space=pltpu.SMEM),
            pl.BlockSpec((1, TD), lambda i: (0, 0)),
            pl.BlockSpec((1, TD), lambda i: (0, 0)),
            pl.BlockSpec((MD, 3 * MD), lambda i: (0, 0)),
            pl.BlockSpec((1, 3 * MD), lambda i: (0, 0)),
            pl.BlockSpec((1, 3 * MD), lambda i: (0, 0)),
            pl.BlockSpec((AUG, ED), lambda i: (0, 0)),
            pl.BlockSpec((AUG, ED), lambda i: (0, 0)),
            pl.BlockSpec((1, ED), lambda i: (0, 0)),
        ],
        out_specs=[
            pl.BlockSpec((BN, ED), lambda i: (i, 0)),
            pl.BlockSpec((BN, ED), lambda i: (i, 0)),
        ],
        out_shape=[
            jax.ShapeDtypeStruct((N, ED), f32),
            jax.ShapeDtypeStruct((N, ED), f32),
        ],
    )(agg_p, cnt_tc, node_features, latest, tw2, tb2, Wi, bi.reshape(1, 3 * MD),
      bh.reshape(1, 3 * MD), Wm_aug, Wo_aug, bout.reshape(1, ED))

    src4 = src.reshape(NW, KB // KC, KC, BB)
    dst4 = dst.reshape(NW, KB // KC, KC, BB)
    agg2_p = _sc_edgeb(paug, c2, src4, dst4)

    logits = pl.pallas_call(
        _final_body,
        grid=(nbn,),
        in_specs=[
            pl.BlockSpec((2, BN, ED), lambda i: (0, i, 0)),
            pl.BlockSpec((BN, ED), lambda i: (i, 0)),
            pl.BlockSpec((ED, ED), lambda i: (0, 0)),
            pl.BlockSpec((ED, ED // 2), lambda i: (0, 0)),
            pl.BlockSpec((1, ED // 2), lambda i: (0, 0)),
            pl.BlockSpec((ED // 2, 2), lambda i: (0, 0)),
            pl.BlockSpec((1, 2), lambda i: (0, 0)),
        ],
        out_specs=pl.BlockSpec((BN, 2), lambda i: (i, 0)),
        out_shape=jax.ShapeDtypeStruct((N, 2), f32),
    )(agg2_p, aout, Wo_agg, Wc1, bc1.reshape(1, ED // 2), Wc2,
      bc2.reshape(1, 2))

    return logits


# speedup vs baseline: 1.1489x; 1.0304x over previous
import functools

import jax
import jax.numpy as jnp
from jax import lax
from jax.experimental import pallas as pl
from jax.experimental.pallas import tpu as pltpu
from jax.experimental.pallas import tpu_sc as plsc

N = 10000
E = 320000
D = 128
DE = 16
TD = 32
MD = 128
ED = 128

NW = 32
EPW = E // NW
B = 80
K = EPW // B
BB = 40
KB = EPW // BB
KC = 50
N_PAD = 10240
RPT = N_PAD // 16

f32 = jnp.float32

_S = (0.9999997069578552, -0.16666577198040858, 0.008332557998290352,
      -0.00019812572236973137, 2.7040473311964398e-06,
      -2.053408005585339e-08)
_TWO_PI_HI = 6.28125
_TWO_PI_LO = 0.0019353071795864769


def _fast_sin(x):
    k = jnp.round(x * (1.0 / 6.283185307179586))
    r = (x - k * _TWO_PI_HI) - k * _TWO_PI_LO
    r2 = r * r
    s = _S[5]
    for c in (_S[4], _S[3], _S[2], _S[1], _S[0]):
        s = s * r2 + c
    return s * r


def _edge_pre_body(et_ref, ea_ref, wef_ref, bmsg_ref, tw_ref, tb_ref,
                   c2_ref, lat_ref, mx_ref):
    i = pl.program_id(0)
    et = et_ref[0, 0, :]
    te = _fast_sin(et[:, None] * tw_ref[0, :][None, :] + tb_ref[0, :][None, :])
    eafeat = jnp.concatenate([ea_ref[...], te], axis=1)
    c2_ref[...] = (jnp.dot(eafeat, wef_ref[...], preferred_element_type=f32)
                   + bmsg_ref[0, :][None, :])

    @pl.when(i == 0)
    def _():
        mx_ref[0] = -jnp.inf

    mx_ref[0] = jnp.maximum(mx_ref[0], jnp.max(et))

    @pl.when(i == pl.num_programs(0) - 1)
    def _():
        lat_ref[0, 0] = mx_ref[0]


def _node_pre_body(x_ref, w1x_ref, b1_ref, p1_ref):
    p1_ref[...] = (jnp.dot(x_ref[...], w1x_ref[...], preferred_element_type=f32)
                   + b1_ref[0, :][None, :])


def _msg_body(g1_ref, ea_ref, src_ref, w1e_ref, w2_ref, b2_ref, out_ref,
              cnt_ref):
    i = pl.program_id(0)
    h = jnp.maximum(
        g1_ref[...] + jnp.dot(ea_ref[...], w1e_ref[...],
                              preferred_element_type=f32), 0.0)
    out_ref[...] = jnp.maximum(
        jnp.dot(h, w2_ref[...], preferred_element_type=f32)
        + b2_ref[0, :][None, :], 0.0)
    srcv = src_ref[0, 0, :]
    g = (srcv[:, None] // 128
         == lax.broadcasted_iota(jnp.int32, (1, N_PAD // 128), 1)).astype(f32)
    o = (srcv[:, None] % 128
         == lax.broadcasted_iota(jnp.int32, (1, 128), 1)).astype(f32)
    contrib = lax.dot_general(g, o, (((0,), (0,)), ((), ())),
                              preferred_element_type=f32)

    @pl.when(i == 0)
    def _():
        cnt_ref[...] = jnp.zeros_like(cnt_ref)

    cnt_ref[...] += contrib


def _gru_body(pa_ref, cnt_ref, x_ref, lat_ref, tw_ref, tb_ref, wi_ref, bi_ref,
              bh_ref, wma_ref, woa_ref, bout_ref, paug_ref, aout_ref):
    cnt = cnt_ref[0, 0, :]
    agg = (pa_ref[0] + pa_ref[1]) / jnp.maximum(cnt, 1.0)[:, None]
    gi = jnp.dot(agg, wi_ref[...], preferred_element_type=f32) + bi_ref[0][None, :]
    bh = bh_ref[0]
    r = jax.nn.sigmoid(gi[:, :MD] + bh[None, :MD])
    z = jax.nn.sigmoid(gi[:, MD:2 * MD] + bh[None, MD:2 * MD])
    nt = jnp.tanh(gi[:, 2 * MD:] + r * bh[None, 2 * MD:])
    mem = (1.0 - z) * nt
    gte = jnp.sin(lat_ref[0, 0] * tw_ref[0] + tb_ref[0])
    aug = jnp.concatenate(
        [x_ref[...], mem,
         jnp.broadcast_to(gte[None, :], (mem.shape[0], TD))], axis=1)
    paug_ref[...] = jnp.dot(aug, wma_ref[...], preferred_element_type=f32)
    aout_ref[...] = (jnp.dot(aug, woa_ref[...], preferred_element_type=f32)
                     + bout_ref[0][None, :])


def _final_body(pa_ref, aout_ref, wog_ref, wc1_ref, bc1_ref, wc2_ref, bc2_ref,
                out_ref):
    agg2 = pa_ref[0] + pa_ref[1]
    x = jnp.maximum(
        aout_ref[...] + jnp.dot(agg2, wog_ref[...], preferred_element_type=f32),
        0.0)
    h2 = jnp.maximum(
        jnp.dot(x, wc1_ref[...], preferred_element_type=f32)
        + bc1_ref[0][None, :], 0.0)
    out_ref[...] = (jnp.dot(h2, wc2_ref[...], preferred_element_type=f32)
                    + bc2_ref[0][None, :])


_MESH = plsc.VectorSubcoreMesh(core_axis_name="c", subcore_axis_name="s")


def _zero_accum(buf, accum, rs):
    nb = buf.shape[0]

    def zb(r, carry):
        for cc8 in range(8):
            buf[r, pl.ds(cc8 * 16, 16)] = jnp.zeros((16,), f32)
        return carry

    lax.fori_loop(0, nb, zb, 0)

    def cp(k, carry):
        off = pl.multiple_of(rs + k * nb, 8)
        pltpu.sync_copy(buf, accum.at[pl.ds(off, nb)])
        return carry

    lax.fori_loop(0, RPT // nb, cp, 0)


@functools.partial(
    pl.kernel, mesh=_MESH,
    out_type=jax.ShapeDtypeStruct((E, D), f32),
    scratch_types=[
        pltpu.VMEM((K, B), jnp.int32),
        pltpu.VMEM((B, D), f32),
        pltpu.VMEM((B, D), f32),
        pltpu.SemaphoreType.DMA,
        pltpu.SemaphoreType.DMA,
    ])
def _sc_gather(table_hbm, idx_hbm, out_hbm, idx_v, rows, rows1, sem, sem1):
    c = lax.axis_index("c")
    s = lax.axis_index("s")
    wid = s * 2 + c
    pltpu.sync_copy(idx_hbm.at[wid], idx_v)
    base = wid * EPW

    def body(t, carry):
        j0 = t * 2
        g0 = pltpu.make_async_copy(table_hbm.at[idx_v.at[j0]], rows, sem)
        g0.start()
        g1 = pltpu.make_async_copy(table_hbm.at[idx_v.at[j0 + 1]], rows1, sem1)
        g1.start()
        g0.wait()
        pltpu.sync_copy(rows, out_hbm.at[pl.ds(pl.multiple_of(base + j0 * B, 16), B)])
        g1.wait()
        pltpu.sync_copy(rows1, out_hbm.at[pl.ds(pl.multiple_of(base + (j0 + 1) * B, 16), B)])
        return carry

    lax.fori_loop(0, K // 2, body, 0)
    jt = K - 1
    pltpu.async_copy(table_hbm.at[idx_v.at[jt]], rows, sem).wait()
    pltpu.sync_copy(rows, out_hbm.at[pl.ds(pl.multiple_of(base + jt * B, 16), B)])


@functools.partial(
    pl.kernel, mesh=_MESH,
    out_type=jax.ShapeDtypeStruct((2, N_PAD, MD), f32),
    scratch_types=[
        pltpu.VMEM((K, B), jnp.int32),
        pltpu.VMEM((B, MD), f32),
        pltpu.VMEM((B, MD), f32),
        pltpu.VMEM_SHARED((N_PAD, MD), f32),
        pltpu.SemaphoreType.DMA,
        pltpu.SemaphoreType.DMA,
    ])
def _sc_scatter(msg_hbm, idx_hbm, out_hbm, idx_v, rows, rows1, accum, sem,
                sem1):
    c = lax.axis_index("c")
    s = lax.axis_index("s")
    wid = s * 2 + c
    rs = s * RPT
    _zero_accum(rows, accum, rs)
    plsc.subcore_barrier()
    pltpu.sync_copy(idx_hbm.at[wid], idx_v)
    base = wid * EPW

    def body(t, carry):
        j0 = t * 2
        off0 = pl.multiple_of(base + j0 * B, 16)
        off1 = pl.multiple_of(base + (j0 + 1) * B, 16)
        g0 = pltpu.make_async_copy(msg_hbm.at[pl.ds(off0, B)], rows, sem)
        g0.start()
        g1 = pltpu.make_async_copy(msg_hbm.at[pl.ds(off1, B)], rows1, sem1)
        g1.start()
        g0.wait()
        pltpu.sync_copy(rows, accum.at[idx_v.at[j0]], add=True)
        g1.wait()
        pltpu.sync_copy(rows1, accum.at[idx_v.at[j0 + 1]], add=True)
        return carry

    lax.fori_loop(0, K // 2, body, 0)
    jt = K - 1
    offt = pl.multiple_of(base + jt * B, 16)
    pltpu.async_copy(msg_hbm.at[pl.ds(offt, B)], rows, sem).wait()
    pltpu.sync_copy(rows, accum.at[idx_v.at[jt]], add=True)
    plsc.subcore_barrier()
    pltpu.sync_copy(accum.at[pl.ds(rs, RPT)], out_hbm.at[c, pl.ds(rs, RPT)])


@functools.partial(
    pl.kernel, mesh=_MESH,
    out_type=jax.ShapeDtypeStruct((2, N_PAD, ED), f32),
    scratch_types=[
        pltpu.VMEM((KC, BB), jnp.int32),
        pltpu.VMEM((KC, BB), jnp.int32),
        pltpu.VMEM((BB, ED), f32),
        pltpu.VMEM((BB, ED), f32),
        pltpu.VMEM((BB, ED), f32),
        pltpu.VMEM((BB, ED), f32),
        pltpu.VMEM_SHARED((N_PAD, ED), f32),
        pltpu.SemaphoreType.DMA,
        pltpu.SemaphoreType.DMA,
        pltpu.SemaphoreType.DMA,
        pltpu.SemaphoreType.DMA,
        pltpu.SemaphoreType.DMA,
        pltpu.SemaphoreType.DMA,
    ])
def _sc_edgeb(paug_hbm, c2_hbm, sidx_hbm, didx_hbm, out_hbm,
              sidx_v, didx_v, gbuf0, gbuf1, cbuf0, cbuf1, accum,
              gsem0, gsem1, csem0, csem1, ssem0, ssem1):
    c = lax.axis_index("c")
    s = lax.axis_index("s")
    wid = s * 2 + c
    rs = s * RPT
    _zero_accum(gbuf0, accum, rs)
    plsc.subcore_barrier()
    base = wid * EPW

    def _relu_add(gbuf, cbuf):
        def ew_row(r, carry):
            for cc8 in range(8):
                col = cc8 * 16
                v = gbuf[r, pl.ds(col, 16)] + cbuf[r, pl.ds(col, 16)]
                gbuf[r, pl.ds(col, 16)] = jnp.maximum(v, 0.0)
            return carry

        lax.fori_loop(0, BB, ew_row, 0)

    def _run(c2_hbm, cbase):
        def outer(jj, carry):
            pltpu.sync_copy(sidx_hbm.at[wid, jj], sidx_v)
            pltpu.sync_copy(didx_hbm.at[wid, jj], didx_v)

            def body(t, carry2):
                j0 = t * 2
                off0 = pl.multiple_of(cbase + (jj * KC + j0) * BB, 8)
                off1 = pl.multiple_of(cbase + (jj * KC + j0 + 1) * BB, 8)
                s0 = pltpu.make_async_copy(gbuf0, accum.at[didx_v.at[j0]],
                                           ssem0)
                s1 = pltpu.make_async_copy(gbuf1, accum.at[didx_v.at[j0 + 1]],
                                           ssem1)

                @pl.when(jj * (KC // 2) + t > 0)
                def _():
                    s0.wait()
                    s1.wait()

                c0 = pltpu.make_async_copy(c2_hbm.at[pl.ds(off0, BB)], cbuf0,
                                           csem0)
                c0.start()
                g0 = pltpu.make_async_copy(paug_hbm.at[sidx_v.at[j0]], gbuf0,
                                           gsem0)
                g0.start()
                c1 = pltpu.make_async_copy(c2_hbm.at[pl.ds(off1, BB)], cbuf1,
                                           csem1)
                c1.start()
                g1 = pltpu.make_async_copy(paug_hbm.at[sidx_v.at[j0 + 1]],
                                           gbuf1, gsem1)
                g1.start()
                g0.wait()
                c0.wait()
                _relu_add(gbuf0, cbuf0)
                pltpu.async_copy(gbuf0, accum.at[didx_v.at[j0]], ssem0,
                                 add=True)
                g1.wait()
                c1.wait()
                _relu_add(gbuf1, cbuf1)
                pltpu.async_copy(gbuf1, accum.at[didx_v.at[j0 + 1]], ssem1,
                                 add=True)
                return carry2

            lax.fori_loop(0, KC // 2, body, 0)
            return carry

        lax.fori_loop(0, KB // KC, outer, 0)
        pltpu.make_async_copy(gbuf0, accum.at[didx_v.at[KC - 2]], ssem0).wait()
        pltpu.make_async_copy(gbuf1, accum.at[didx_v.at[KC - 1]], ssem1).wait()

    _run(c2_hbm, base)
    plsc.subcore_barrier()
    pltpu.sync_copy(accum.at[pl.ds(rs, RPT)], out_hbm.at[c, pl.ds(rs, RPT)])


def kernel(node_features, edge_index, edge_attr, edge_times, time_w, time_b,
           W1, b1, W2, b2, Wi, bi, Wh, bh, Wmsg, bmsg, Wout, bout,
           Wc1, bc1, Wc2, bc2):
    del Wh
    BE = 2000
    BN = 2000
    nbe = E // BE
    nbn = N // BN
    AUG = D + MD + TD

    src = edge_index[0].astype(jnp.int32)
    dst = edge_index[1].astype(jnp.int32)
    src3 = src.reshape(NW, K, B)
    dst3 = dst.reshape(NW, K, B)
    src_tc = src.reshape(nbe, 1, BE)
    et3 = edge_times.reshape(nbe, 1, BE)
    tw2 = time_w.reshape(1, TD)
    tb2 = time_b.reshape(1, TD)

    W1x = W1[:D]
    W1e = W1[D + MD:]
    Wm_aug = Wmsg[:AUG]
    Wm_ef = Wmsg[AUG:]
    Wo_aug = Wout[:AUG]
    Wo_agg = Wout[AUG:]

    c2, latest = pl.pallas_call(
        _edge_pre_body,
        grid=(nbe,),
        in_specs=[
            pl.BlockSpec((1, 1, BE), lambda i: (i, 0, 0)),
            pl.BlockSpec((BE, DE), lambda i: (i, 0)),
            pl.BlockSpec((DE + TD, ED), lambda i: (0, 0)),
            pl.BlockSpec((1, ED), lambda i: (0, 0)),
            pl.BlockSpec((1, TD), lambda i: (0, 0)),
            pl.BlockSpec((1, TD), lambda i: (0, 0)),
        ],
        out_specs=[
            pl.BlockSpec((BE, ED), lambda i: (i, 0)),
            pl.BlockSpec(memory_space=pltpu.SMEM),
        ],
        out_shape=[
            jax.ShapeDtypeStruct((E, ED), f32),
            jax.ShapeDtypeStruct((1, 1), f32),
        ],
        scratch_shapes=[pltpu.SMEM((1,), f32)],
    )(et3, edge_attr, Wm_ef, bmsg.reshape(1, ED), tw2, tb2)

    p1 = pl.pallas_call(
        _node_pre_body,
        grid=(nbn,),
        in_specs=[
            pl.BlockSpec((BN, D), lambda i: (i, 0)),
            pl.BlockSpec((D, ED), lambda i: (0, 0)),
            pl.BlockSpec((1, ED), lambda i: (0, 0)),
        ],
        out_specs=pl.BlockSpec((BN, ED), lambda i: (i, 0)),
        out_shape=jax.ShapeDtypeStruct((N, ED), f32),
    )(node_features, W1x, b1.reshape(1, ED))

    g1 = _sc_gather(p1, src3)

    msg, cnt_hist = pl.pallas_call(
        _msg_body,
        grid=(nbe,),
        in_specs=[
            pl.BlockSpec((BE, ED), lambda i: (i, 0)),
            pl.BlockSpec((BE, DE), lambda i: (i, 0)),
            pl.BlockSpec((1, 1, BE), lambda i: (i, 0, 0)),
            pl.BlockSpec((DE, ED), lambda i: (0, 0)),
            pl.BlockSpec((ED, MD), lambda i: (0, 0)),
            pl.BlockSpec((1, MD), lambda i: (0, 0)),
        ],
        out_specs=[
            pl.BlockSpec((BE, MD), lambda i: (i, 0)),
            pl.BlockSpec((N_PAD // 128, 128), lambda i: (0, 0)),
        ],
        out_shape=[
            jax.ShapeDtypeStruct((E, MD), f32),
            jax.ShapeDtypeStruct((N_PAD // 128, 128), f32),
        ],
    )(g1, edge_attr, src_tc, W1e, W2, b2.reshape(1, MD))
    cnt_tc = cnt_hist.reshape(N_PAD)[:N].reshape(nbn, 1, BN)

    agg_p = _sc_scatter(msg, src3)

    paug, aout = pl.pallas_call(
        _gru_body,
        grid=(nbn,),
        in_specs=[
            pl.BlockSpec((2, BN, MD), lambda i: (0, i, 0)),
            pl.BlockSpec((1, 1, BN), lambda i: (i, 0, 0)),
            pl.BlockSpec((BN, D), lambda i: (i, 0)),
            pl.BlockSpec(memory_space=pltpu.SMEM),
            pl.BlockSpec((1, TD), lambda i: (0, 0)),
            pl.BlockSpec((1, TD), lambda i: (0, 0)),
            pl.BlockSpec((MD, 3 * MD), lambda i: (0, 0)),
            pl.BlockSpec((1, 3 * MD), lambda i: (0, 0)),
            pl.BlockSpec((1, 3 * MD), lambda i: (0, 0)),
            pl.BlockSpec((AUG, ED), lambda i: (0, 0)),
            pl.BlockSpec((AUG, ED), lambda i: (0, 0)),
            pl.BlockSpec((1, ED), lambda i: (0, 0)),
        ],
        out_specs=[
            pl.BlockSpec((BN, ED), lambda i: (i, 0)),
            pl.BlockSpec((BN, ED), lambda i: (i, 0)),
        ],
        out_shape=[
            jax.ShapeDtypeStruct((N, ED), f32),
            jax.ShapeDtypeStruct((N, ED), f32),
        ],
    )(agg_p, cnt_tc, node_features, latest, tw2, tb2, Wi, bi.reshape(1, 3 * MD),
      bh.reshape(1, 3 * MD), Wm_aug, Wo_aug, bout.reshape(1, ED))

    src4 = src.reshape(NW, KB // KC, KC, BB)
    dst4 = dst.reshape(NW, KB // KC, KC, BB)
    agg2_p = _sc_edgeb(paug, c2, src4, dst4)

    logits = pl.pallas_call(
        _final_body,
        grid=(nbn,),
        in_specs=[
            pl.BlockSpec((2, BN, ED), lambda i: (0, i, 0)),
            pl.BlockSpec((BN, ED), lambda i: (i, 0)),
            pl.BlockSpec((ED, ED), lambda i: (0, 0)),
            pl.BlockSpec((ED, ED // 2), lambda i: (0, 0)),
            pl.BlockSpec((1, ED // 2), lambda i: (0, 0)),
            pl.BlockSpec((ED // 2, 2), lambda i: (0, 0)),
            pl.BlockSpec((1, 2), lambda i: (0, 0)),
        ],
        out_specs=pl.BlockSpec((BN, 2), lambda i: (i, 0)),
        out_shape=jax.ShapeDtypeStruct((N, 2), f32),
    )(agg2_p, aout, Wo_agg, Wc1, bc1.reshape(1, ED // 2), Wc2,
      bc2.reshape(1, 2))

    return logits
